# baseline (speedup 1.0000x reference)
import functools

import jax
import jax.numpy as jnp
from jax import lax
from jax.experimental import pallas as pl
from jax.experimental.pallas import tpu as pltpu
from jax.experimental.pallas import tpu_sc as plsc

_B, _N, _D = 16, 576, 64
_K = 1024
_BETA = 0.25
_M = _B * _N
_R = 512
_GRID = _M // _R

_NUM_CORES = 2
_NUM_SUBCORES = 16
_NW = _NUM_CORES * _NUM_SUBCORES
_RPW = _M // _NW


def _tc_body(z_ref, w_ref, idx_ref, loss_ref):
    i = pl.program_id(0)
    z = z_ref[...]
    w = w_ref[...]
    z_sq = jnp.sum(z * z, axis=1, keepdims=True)
    e_sq = jnp.sum(w * w, axis=1)
    dot = lax.dot_general(z, w, (((1,), (1,)), ((), ())))
    d = z_sq + e_sq[None, :] - 2.0 * dot
    m = jnp.min(d, axis=1, keepdims=True)
    cols = lax.broadcasted_iota(jnp.int32, (_R, _K), 1)
    idx = jnp.min(jnp.where(d == m, cols, _K), axis=1)
    idx_ref[...] = idx

    @pl.when(i == 0)
    def _init():
        loss_ref[...] = jnp.zeros((1, 1), jnp.float32)

    loss_ref[...] += jnp.sum(m).reshape(1, 1)

    @pl.when(i == _GRID - 1)
    def _finalize():
        s = loss_ref[...] * (1.0 / float(_M * _D))
        loss_ref[...] = s + _BETA * s


def _tc_argmin(zf, w):
    return pl.pallas_call(
        _tc_body,
        grid=(_GRID,),
        in_specs=[
            pl.BlockSpec((_R, _D), lambda i: (i, 0)),
            pl.BlockSpec((_K, _D), lambda i: (0, 0)),
        ],
        out_specs=[
            pl.BlockSpec((_R,), lambda i: (i,)),
            pl.BlockSpec((1, 1), lambda i: (0, 0)),
        ],
        out_shape=[
            jax.ShapeDtypeStruct((_M,), jnp.int32),
            jax.ShapeDtypeStruct((1, 1), jnp.float32),
        ],
    )(zf, w)


def _sc_gather_body(table_hbm, idx_hbm, out_hbm, idx_v, rows_v, sem):
    wid = lax.axis_index("s") * _NUM_CORES + lax.axis_index("c")
    base = wid * _RPW
    pltpu.sync_copy(idx_hbm.at[pl.ds(base, _RPW)], idx_v)
    pltpu.async_copy(table_hbm.at[idx_v], rows_v, sem).wait()
    pltpu.sync_copy(rows_v, out_hbm.at[pl.ds(base, _RPW)])


@functools.cache
def _sc_gather():
    return pl.kernel(
        _sc_gather_body,
        out_type=jax.ShapeDtypeStruct((_M, _D), jnp.float32),
        mesh=plsc.VectorSubcoreMesh(
            core_axis_name="c", subcore_axis_name="s",
            num_cores=_NUM_CORES, num_subcores=_NUM_SUBCORES),
        scratch_types=[
            pltpu.VMEM((_RPW,), jnp.int32),
            pltpu.VMEM((_RPW, _D), jnp.float32),
            pltpu.SemaphoreType.DMA,
        ],
        compiler_params=pltpu.CompilerParams(use_tc_tiling_on_sc=False),
    )


def kernel(z, W):
    zf = z.reshape(_M, _D)
    idx, loss = _tc_argmin(zf, W)
    q = _sc_gather()(W, idx)
    return (q.reshape(_B, _N, _D), loss[0, 0], idx.reshape(_B, _N))


# speedup vs baseline: 1.0853x; 1.0853x over previous
import functools

import jax
import jax.numpy as jnp
from jax import lax
from jax.experimental import pallas as pl
from jax.experimental.pallas import tpu as pltpu
from jax.experimental.pallas import tpu_sc as plsc

_B, _N, _D = 16, 576, 64
_K = 1024
_BETA = 0.25
_M = _B * _N
_R = 512
_GRID = _M // _R

_NUM_CORES = 2
_NUM_SUBCORES = 16
_NW = _NUM_CORES * _NUM_SUBCORES
_RPW = _M // _NW


def _tc_body(z_ref, w_ref, idx_ref, loss_ref):
    i = pl.program_id(0)
    z = z_ref[...]
    w = w_ref[...]
    z_sq = jnp.sum(z * z, axis=1, keepdims=True)
    e_sq = jnp.sum(w * w, axis=1)
    dot = lax.dot_general(z, w, (((1,), (1,)), ((), ())))
    d = z_sq + e_sq[None, :] - 2.0 * dot
    m = jnp.min(d, axis=1, keepdims=True)
    cols = lax.broadcasted_iota(jnp.int32, (1, _K), 1).astype(jnp.float32)
    idxf = jnp.min(jnp.where(d == m, cols, float(_K)), axis=1)
    idx_ref[...] = idxf.astype(jnp.int32)

    @pl.when(i == 0)
    def _init():
        loss_ref[...] = jnp.zeros((1, 1), jnp.float32)

    loss_ref[...] += jnp.sum(m).reshape(1, 1)

    @pl.when(i == _GRID - 1)
    def _finalize():
        s = loss_ref[...] * (1.0 / float(_M * _D))
        loss_ref[...] = s + _BETA * s


def _tc_argmin(zf, w):
    return pl.pallas_call(
        _tc_body,
        grid=(_GRID,),
        in_specs=[
            pl.BlockSpec((_R, _D), lambda i: (i, 0)),
            pl.BlockSpec((_K, _D), lambda i: (0, 0)),
        ],
        out_specs=[
            pl.BlockSpec((_R,), lambda i: (i,)),
            pl.BlockSpec((1, 1), lambda i: (0, 0)),
        ],
        out_shape=[
            jax.ShapeDtypeStruct((_M,), jnp.int32),
            jax.ShapeDtypeStruct((1, 1), jnp.float32),
        ],
    )(zf, w)


def _sc_gather_body(table_hbm, idx_hbm, out_hbm, idx_v, rows_v, sem):
    wid = lax.axis_index("s") * _NUM_CORES + lax.axis_index("c")
    base = wid * _RPW
    pltpu.sync_copy(idx_hbm.at[pl.ds(base, _RPW)], idx_v)
    pltpu.async_copy(table_hbm.at[idx_v], rows_v, sem).wait()
    pltpu.sync_copy(rows_v, out_hbm.at[pl.ds(base, _RPW)])


@functools.cache
def _sc_gather():
    return pl.kernel(
        _sc_gather_body,
        out_type=jax.ShapeDtypeStruct((_M, _D), jnp.float32),
        mesh=plsc.VectorSubcoreMesh(
            core_axis_name="c", subcore_axis_name="s",
            num_cores=_NUM_CORES, num_subcores=_NUM_SUBCORES),
        scratch_types=[
            pltpu.VMEM((_RPW,), jnp.int32),
            pltpu.VMEM((_RPW, _D), jnp.float32),
            pltpu.SemaphoreType.DMA,
        ],
        compiler_params=pltpu.CompilerParams(use_tc_tiling_on_sc=False),
    )


def kernel(z, W):
    zf = z.reshape(_M, _D)
    idx, loss = _tc_argmin(zf, W)
    q = _sc_gather()(W, idx)
    return (q.reshape(_B, _N, _D), loss[0, 0], idx.reshape(_B, _N))


# speedup vs baseline: 1.0904x; 1.0047x over previous
import functools

import jax
import jax.numpy as jnp
from jax import lax
from jax.experimental import pallas as pl
from jax.experimental.pallas import tpu as pltpu
from jax.experimental.pallas import tpu_sc as plsc

_B, _N, _D = 16, 576, 64
_K = 1024
_BETA = 0.25
_M = _B * _N
_R = 1024
_GRID = _M // _R

_NUM_CORES = 2
_NUM_SUBCORES = 16
_NW = _NUM_CORES * _NUM_SUBCORES
_RPW = _M // _NW


def _tc_body(z_ref, w_ref, idx_ref, loss_ref):
    i = pl.program_id(0)
    z = z_ref[...]
    w = w_ref[...]
    z_sq = jnp.sum(z * z, axis=1, keepdims=True)
    e_sq = jnp.sum(w * w, axis=1)
    dot = lax.dot_general(z, w, (((1,), (1,)), ((), ())))
    d = z_sq + e_sq[None, :] - 2.0 * dot
    m = jnp.min(d, axis=1, keepdims=True)
    cols = lax.broadcasted_iota(jnp.int32, (1, _K), 1).astype(jnp.float32)
    idxf = jnp.min(jnp.where(d == m, cols, float(_K)), axis=1)
    idx_ref[...] = idxf.astype(jnp.int32)

    @pl.when(i == 0)
    def _init():
        loss_ref[...] = jnp.zeros((1, 1), jnp.float32)

    loss_ref[...] += jnp.sum(m).reshape(1, 1)

    @pl.when(i == _GRID - 1)
    def _finalize():
        s = loss_ref[...] * (1.0 / float(_M * _D))
        loss_ref[...] = s + _BETA * s


def _tc_argmin(zf, w):
    return pl.pallas_call(
        _tc_body,
        grid=(_GRID,),
        in_specs=[
            pl.BlockSpec((_R, _D), lambda i: (i, 0)),
            pl.BlockSpec((_K, _D), lambda i: (0, 0)),
        ],
        out_specs=[
            pl.BlockSpec((_R,), lambda i: (i,)),
            pl.BlockSpec((1, 1), lambda i: (0, 0)),
        ],
        out_shape=[
            jax.ShapeDtypeStruct((_M,), jnp.int32),
            jax.ShapeDtypeStruct((1, 1), jnp.float32),
        ],
    )(zf, w)


def _sc_gather_body(table_hbm, idx_hbm, out_hbm, idx_v, rows_v, sem):
    wid = lax.axis_index("s") * _NUM_CORES + lax.axis_index("c")
    base = wid * _RPW
    pltpu.sync_copy(idx_hbm.at[pl.ds(base, _RPW)], idx_v)
    pltpu.async_copy(table_hbm.at[idx_v], rows_v, sem).wait()
    pltpu.sync_copy(rows_v, out_hbm.at[pl.ds(base, _RPW)])


@functools.cache
def _sc_gather():
    return pl.kernel(
        _sc_gather_body,
        out_type=jax.ShapeDtypeStruct((_M, _D), jnp.float32),
        mesh=plsc.VectorSubcoreMesh(
            core_axis_name="c", subcore_axis_name="s",
            num_cores=_NUM_CORES, num_subcores=_NUM_SUBCORES),
        scratch_types=[
            pltpu.VMEM((_RPW,), jnp.int32),
            pltpu.VMEM((_RPW, _D), jnp.float32),
            pltpu.SemaphoreType.DMA,
        ],
        compiler_params=pltpu.CompilerParams(use_tc_tiling_on_sc=False),
    )


def kernel(z, W):
    zf = z.reshape(_M, _D)
    idx, loss = _tc_argmin(zf, W)
    q = _sc_gather()(W, idx)
    return (q.reshape(_B, _N, _D), loss[0, 0], idx.reshape(_B, _N))


# speedup vs baseline: 1.4492x; 1.3290x over previous
import functools

import jax
import jax.numpy as jnp
from jax import lax
from jax.experimental import pallas as pl
from jax.experimental.pallas import tpu as pltpu
from jax.experimental.pallas import tpu_sc as plsc

_B, _N, _D = 16, 576, 64
_K = 1024
_BETA = 0.25
_M = _B * _N
_R = 1024
_GRID = _M // _R

_NUM_CORES = 2
_NUM_SUBCORES = 16
_NW = _NUM_CORES * _NUM_SUBCORES
_RPW = _M // _NW


def _tc_body(z_ref, w_ref, idx_ref, loss_ref, q_ref):
    i = pl.program_id(0)
    z = z_ref[...]
    w = w_ref[...]
    z_sq = jnp.sum(z * z, axis=1, keepdims=True)
    e_sq = jnp.sum(w * w, axis=1)
    dot = lax.dot_general(z, w, (((1,), (1,)), ((), ())))
    d = z_sq + e_sq[None, :] - 2.0 * dot
    m = jnp.min(d, axis=1, keepdims=True)
    cols = lax.broadcasted_iota(jnp.int32, (1, _K), 1).astype(jnp.float32)
    idxf = jnp.min(jnp.where(d == m, cols, float(_K)), axis=1)
    idx_ref[...] = idxf.astype(jnp.int32)
    oh = (cols == idxf[:, None]).astype(jnp.float32)
    q_ref[...] = lax.dot_general(oh, w, (((1,), (0,)), ((), ())))

    @pl.when(i == 0)
    def _init():
        loss_ref[...] = jnp.zeros((1, 1), jnp.float32)

    loss_ref[...] += jnp.sum(m).reshape(1, 1)

    @pl.when(i == _GRID - 1)
    def _finalize():
        s = loss_ref[...] * (1.0 / float(_M * _D))
        loss_ref[...] = s + _BETA * s


def _tc_argmin(zf, w):
    return pl.pallas_call(
        _tc_body,
        grid=(_GRID,),
        in_specs=[
            pl.BlockSpec((_R, _D), lambda i: (i, 0)),
            pl.BlockSpec((_K, _D), lambda i: (0, 0)),
        ],
        out_specs=[
            pl.BlockSpec((_R,), lambda i: (i,)),
            pl.BlockSpec((1, 1), lambda i: (0, 0)),
            pl.BlockSpec((_R, _D), lambda i: (i, 0)),
        ],
        out_shape=[
            jax.ShapeDtypeStruct((_M,), jnp.int32),
            jax.ShapeDtypeStruct((1, 1), jnp.float32),
            jax.ShapeDtypeStruct((_M, _D), jnp.float32),
        ],
    )(zf, w)


def _sc_gather_body(table_hbm, idx_hbm, out_hbm, idx_v, rows_v, sem):
    wid = lax.axis_index("s") * _NUM_CORES + lax.axis_index("c")
    base = wid * _RPW
    pltpu.sync_copy(idx_hbm.at[pl.ds(base, _RPW)], idx_v)
    pltpu.async_copy(table_hbm.at[idx_v], rows_v, sem).wait()
    pltpu.sync_copy(rows_v, out_hbm.at[pl.ds(base, _RPW)])


@functools.cache
def _sc_gather():
    return pl.kernel(
        _sc_gather_body,
        out_type=jax.ShapeDtypeStruct((_M, _D), jnp.float32),
        mesh=plsc.VectorSubcoreMesh(
            core_axis_name="c", subcore_axis_name="s",
            num_cores=_NUM_CORES, num_subcores=_NUM_SUBCORES),
        scratch_types=[
            pltpu.VMEM((_RPW,), jnp.int32),
            pltpu.VMEM((_RPW, _D), jnp.float32),
            pltpu.SemaphoreType.DMA,
        ],
        compiler_params=pltpu.CompilerParams(use_tc_tiling_on_sc=False),
    )


def kernel(z, W):
    zf = z.reshape(_M, _D)
    idx, loss, q = _tc_argmin(zf, W)
    return (q.reshape(_B, _N, _D), loss[0, 0], idx.reshape(_B, _N))
